# baseline (speedup 1.0000x reference)
import functools

import jax
import jax.numpy as jnp
from jax import lax
from jax.experimental import pallas as pl
from jax.experimental.pallas import tpu as pltpu
from jax.experimental.pallas import tpu_sc as plsc

N = 50000
E = 800000
G = 64
GP = 80
NP = 50176
TILE_N = NP // 16
EB = 6272
EBLK = 128
EPAD = EB * EBLK
NBT_ALL = EB // 16
NBT_SPL = EB // 32
DPIPE = 4
RB = 1568
NBLK = NP // RB
F32 = jnp.float32

_MESH = plsc.VectorSubcoreMesh(core_axis_name="c", subcore_axis_name="s")
_SC_PARAMS = pltpu.CompilerParams(needs_layout_passes=False,
                                  use_tc_tiling_on_sc=False)
_NSD = jax.ShapeDtypeStruct((NP, 32), F32)


def _edge_pipeline(idx_start, idx_wait, work, drain_one, nblocks):
    ngrp = nblocks // DPIPE
    for b in range(DPIPE):
        idx_start(b, b)

    def grp(g, carry):
        for b in range(DPIPE):
            idx_wait(b)
            work(b)

        def drain(b, carry2):
            drain_one(b)

            @pl.when(g < ngrp - 1)
            def _():
                idx_start(g * DPIPE + DPIPE + b, b)
            return carry2
        lax.fori_loop(0, DPIPE, drain, 0)
        return carry
    lax.fori_loop(0, ngrp, grp, 0)


@functools.partial(
    pl.kernel,
    out_type=[_NSD, _NSD],
    mesh=_MESH,
    compiler_params=_SC_PARAMS,
    scratch_types=[
        pltpu.VMEM((DPIPE, EBLK), jnp.int32),
        pltpu.VMEM((EBLK, 32), F32),
        pltpu.VMEM_SHARED((NP, 32), F32),
        pltpu.SemaphoreType.DMA,
        pltpu.SemaphoreType.DMA,
    ],
)
def _deg_g(dst_hbm, onesb_hbm, zblk_hbm, oa, ob, dstb, ones, acc, isem, ssem):
    c = lax.axis_index("c")
    s = lax.axis_index("s")
    pltpu.sync_copy(onesb_hbm, ones)
    pltpu.sync_copy(zblk_hbm, acc.at[pl.ds(s * TILE_N, TILE_N)])
    plsc.subcore_barrier()

    def idx_start(j, b):
        pltpu.async_copy(
            dst_hbm.at[pl.ds(((c * 16 + s) * NBT_SPL + j) * EBLK, EBLK)],
            dstb.at[b], isem)

    def idx_wait(b):
        pltpu.make_async_copy(dst_hbm.at[pl.ds(0, EBLK)],
                              dstb.at[b], isem).wait()

    def work(b):
        pltpu.async_copy(ones, acc.at[dstb.at[b]], ssem, add=True)

    def drain_one(b):
        pltpu.make_async_copy(onesb_hbm, ones, ssem).wait()

    _edge_pipeline(idx_start, idx_wait, work, drain_one, NBT_SPL)
    plsc.subcore_barrier()
    for cc, out in ((0, oa), (1, ob)):
        @pl.when(c == cc)
        def _(out=out):
            pltpu.sync_copy(acc.at[pl.ds(s * TILE_N, TILE_N)],
                            out.at[pl.ds(s * TILE_N, TILE_N)])


@functools.partial(
    pl.kernel,
    out_type=[_NSD, _NSD, _NSD, _NSD],
    mesh=_MESH,
    compiler_params=_SC_PARAMS,
    scratch_types=[
        pltpu.VMEM((DPIPE, EBLK), jnp.int32),
        pltpu.VMEM((DPIPE, EBLK), jnp.int32),
        pltpu.VMEM((DPIPE, EBLK, 32), F32),
        pltpu.VMEM_SHARED((NP, 32), F32),
        pltpu.SemaphoreType.DMA,
        pltpu.SemaphoreType.DMA,
        pltpu.SemaphoreType.DMA,
    ],
)
def _agg4_g(y0_hbm, y1_hbm, y2_hbm, y3_hbm, src_hbm, dst_hbm, zblk_hbm,
            o0, o1, o2, o3, srcb, dstb, rows, acc, isem, gsem, ssem):
    c = lax.axis_index("c")
    s = lax.axis_index("s")
    ys = (y0_hbm, y1_hbm, y2_hbm, y3_hbm)
    outs = (o0, o1, o2, o3)

    def idx_start(j, b):
        base = (s * NBT_ALL + j) * EBLK
        pltpu.async_copy(src_hbm.at[pl.ds(base, EBLK)], srcb.at[b], isem)
        pltpu.async_copy(dst_hbm.at[pl.ds(base, EBLK)], dstb.at[b], isem)

    def idx_wait(b):
        pltpu.make_async_copy(src_hbm.at[pl.ds(0, EBLK)],
                              srcb.at[b], isem).wait()
        pltpu.make_async_copy(dst_hbm.at[pl.ds(0, EBLK)],
                              dstb.at[b], isem).wait()

    def drain_one(b):
        pltpu.make_async_copy(y0_hbm.at[pl.ds(0, EBLK)],
                              rows.at[b], ssem).wait()

    for q in range(4):
        @pl.when(c == q // 2)
        def _(q=q):
            y = ys[q]
            pltpu.sync_copy(zblk_hbm, acc.at[pl.ds(s * TILE_N, TILE_N)])
            plsc.subcore_barrier()

            def work(b):
                pltpu.async_copy(y.at[srcb.at[b]], rows.at[b], gsem)

                def scat(bb, carry):
                    pltpu.make_async_copy(y.at[pl.ds(0, EBLK)],
                                          rows.at[bb], gsem).wait()
                    pltpu.async_copy(rows.at[bb], acc.at[dstb.at[bb]], ssem,
                                     add=True)
                    return carry
                if b == DPIPE - 1:
                    lax.fori_loop(0, DPIPE, scat, 0)

            _edge_pipeline(idx_start, idx_wait, work, drain_one, NBT_ALL)
            plsc.subcore_barrier()
            pltpu.sync_copy(acc.at[pl.ds(s * TILE_N, TILE_N)],
                            outs[q].at[pl.ds(s * TILE_N, TILE_N)])
            plsc.subcore_barrier()


@functools.partial(
    pl.kernel,
    out_type=[_NSD, _NSD],
    mesh=_MESH,
    compiler_params=_SC_PARAMS,
    scratch_types=[
        pltpu.VMEM((DPIPE, EBLK), jnp.int32),
        pltpu.VMEM((DPIPE, EBLK), jnp.int32),
        pltpu.VMEM((DPIPE, EBLK, 32), F32),
        pltpu.VMEM_SHARED((NP, 32), F32),
        pltpu.SemaphoreType.DMA,
        pltpu.SemaphoreType.DMA,
        pltpu.SemaphoreType.DMA,
    ],
)
def _agg2_g(ylo_hbm, yhi_hbm, src_hbm, dst_hbm, zblk_hbm, olo, ohi,
            srcb, dstb, rows, acc, isem, gsem, ssem):
    c = lax.axis_index("c")
    s = lax.axis_index("s")
    pltpu.sync_copy(zblk_hbm, acc.at[pl.ds(s * TILE_N, TILE_N)])
    plsc.subcore_barrier()

    def idx_start(j, b):
        base = (s * NBT_ALL + j) * EBLK
        pltpu.async_copy(src_hbm.at[pl.ds(base, EBLK)], srcb.at[b], isem)
        pltpu.async_copy(dst_hbm.at[pl.ds(base, EBLK)], dstb.at[b], isem)

    def idx_wait(b):
        pltpu.make_async_copy(src_hbm.at[pl.ds(0, EBLK)],
                              srcb.at[b], isem).wait()
        pltpu.make_async_copy(dst_hbm.at[pl.ds(0, EBLK)],
                              dstb.at[b], isem).wait()

    def drain_one(b):
        pltpu.make_async_copy(ylo_hbm.at[pl.ds(0, EBLK)],
                              rows.at[b], ssem).wait()

    for h, y, out in ((0, ylo_hbm, olo), (1, yhi_hbm, ohi)):
        @pl.when(c == h)
        def _(y=y, out=out):
            def work(b):
                pltpu.async_copy(y.at[srcb.at[b]], rows.at[b], gsem)

                def scat(bb, carry):
                    pltpu.make_async_copy(y.at[pl.ds(0, EBLK)],
                                          rows.at[bb], gsem).wait()
                    pltpu.async_copy(rows.at[bb], acc.at[dstb.at[bb]], ssem,
                                     add=True)
                    return carry
                if b == DPIPE - 1:
                    lax.fori_loop(0, DPIPE, scat, 0)

            _edge_pipeline(idx_start, idx_wait, work, drain_one, NBT_ALL)
            plsc.subcore_barrier()
            pltpu.sync_copy(acc.at[pl.ds(s * TILE_N, TILE_N)],
                            out.at[pl.ds(s * TILE_N, TILE_N)])


@functools.partial(
    pl.kernel,
    out_type=[_NSD, _NSD],
    mesh=_MESH,
    compiler_params=_SC_PARAMS,
    scratch_types=[
        pltpu.VMEM((DPIPE, EBLK), jnp.int32),
        pltpu.VMEM((DPIPE, EBLK), jnp.int32),
        pltpu.VMEM((DPIPE, EBLK, 32), F32),
        pltpu.VMEM_SHARED((NP, 32), F32),
        pltpu.SemaphoreType.DMA,
        pltpu.SemaphoreType.DMA,
        pltpu.SemaphoreType.DMA,
    ],
)
def _agg1_g(y_hbm, src_hbm, dst_hbm, zblk_hbm, oa, ob,
            srcb, dstb, rows, acc, isem, gsem, ssem):
    c = lax.axis_index("c")
    s = lax.axis_index("s")
    pltpu.sync_copy(zblk_hbm, acc.at[pl.ds(s * TILE_N, TILE_N)])
    plsc.subcore_barrier()

    def idx_start(j, b):
        base = ((c * 16 + s) * NBT_SPL + j) * EBLK
        pltpu.async_copy(src_hbm.at[pl.ds(base, EBLK)], srcb.at[b], isem)
        pltpu.async_copy(dst_hbm.at[pl.ds(base, EBLK)], dstb.at[b], isem)

    def idx_wait(b):
        pltpu.make_async_copy(src_hbm.at[pl.ds(0, EBLK)],
                              srcb.at[b], isem).wait()
        pltpu.make_async_copy(dst_hbm.at[pl.ds(0, EBLK)],
                              dstb.at[b], isem).wait()

    def work(b):
        pltpu.async_copy(y_hbm.at[srcb.at[b]], rows.at[b], gsem)

        def scat(bb, carry):
            pltpu.make_async_copy(y_hbm.at[pl.ds(0, EBLK)],
                                  rows.at[bb], gsem).wait()
            pltpu.async_copy(rows.at[bb], acc.at[dstb.at[bb]], ssem, add=True)
            return carry
        if b == DPIPE - 1:
            lax.fori_loop(0, DPIPE, scat, 0)

    def drain_one(b):
        pltpu.make_async_copy(y_hbm.at[pl.ds(0, EBLK)],
                              rows.at[b], ssem).wait()

    _edge_pipeline(idx_start, idx_wait, work, drain_one, NBT_SPL)
    plsc.subcore_barrier()
    for cc, out in ((0, oa), (1, ob)):
        @pl.when(c == cc)
        def _(out=out):
            pltpu.sync_copy(acc.at[pl.ds(s * TILE_N, TILE_N)],
                            out.at[pl.ds(s * TILE_N, TILE_N)])


_PR = NP // 8
_CHZ = 448
_NCHZ = _PR // _CHZ


@functools.partial(
    pl.kernel,
    out_type=jax.ShapeDtypeStruct((G,), F32),
    mesh=_MESH,
    compiler_params=_SC_PARAMS,
    scratch_types=[
        pltpu.VMEM((_CHZ, 32), F32),
        pltpu.VMEM((_PR,), jnp.int32),
        pltpu.VMEM((GP,), F32),
        pltpu.VMEM((GP,), F32),
        pltpu.VMEM((16 * 2 * GP,), F32),
        pltpu.VMEM((G,), F32),
        pltpu.VMEM((16,), F32),
        pltpu.VMEM_SHARED((16 * 2 * GP,), F32),
    ],
)
def _pool_kernel(zq_hbm, zr_hbm, bq_hbm, br_hbm, fcb_hbm, out_hbm,
                 zbuf, bbuf, accl, cntl, slab, obuf, fbuf, stage):
    c = lax.axis_index("c")
    s = lax.axis_index("s")
    gg = lax.rem(s, 2)
    ci = lax.div(s, 2)
    zero16 = jnp.zeros((16,), F32)
    ones16 = jnp.ones((16,), F32)
    iota16 = lax.iota(jnp.int32, 16)
    zero16i = jnp.zeros((16,), jnp.int32)
    for gch, bh in ((0, bq_hbm), (1, br_hbm)):
        @pl.when(gg == gch)
        def _(bh=bh):
            pltpu.sync_copy(bh.at[pl.ds(ci * _PR, _PR)], bbuf)
    pltpu.sync_copy(fcb_hbm, fbuf)
    for t in range(GP // 16):
        accl[pl.ds(t * 16, 16)] = zero16
        cntl[pl.ds(t * 16, 16)] = zero16

    def chunk(ch, carry):
        for gch, zh in ((0, zq_hbm), (1, zr_hbm)):
            @pl.when(gg == gch)
            def _(zh=zh):
                pltpu.sync_copy(zh.at[pl.ds(ci * _PR + ch * _CHZ, _CHZ)], zbuf)

        def it(k, carry2):
            b16 = bbuf[pl.ds(ch * _CHZ + k * 16, 16)]
            rows16 = iota16 + k * 16
            z16 = plsc.load_gather(zbuf, [rows16, zero16i])
            plsc.addupdate_scatter(accl, [b16], z16)
            plsc.addupdate_scatter(cntl, [b16], ones16)
            return carry2
        lax.fori_loop(0, _CHZ // 16, it, 0)
        return carry
    lax.fori_loop(0, _NCHZ, chunk, 0)

    pltpu.sync_copy(accl, stage.at[pl.ds(s * 2 * GP, GP)])
    pltpu.sync_copy(cntl, stage.at[pl.ds(s * 2 * GP + GP, GP)])
    plsc.subcore_barrier()

    @pl.when(jnp.logical_and(c == 0, s == 0))
    def _():
        pltpu.sync_copy(stage, slab)
        fcb16 = fbuf[...]
        for j in range(G // 16):
            def sl(r):
                return pl.ds(r * 2 * GP + j * 16, 16)

            def slc(r):
                return pl.ds(r * 2 * GP + GP + j * 16, 16)
            aq = slab[sl(0)]
            cq = slab[slc(0)]
            ar = slab[sl(1)]
            cr = slab[slc(1)]
            for r in range(2, 16, 2):
                aq = aq + slab[sl(r)]
                cq = cq + slab[slc(r)]
                ar = ar + slab[sl(r + 1)]
                cr = cr + slab[slc(r + 1)]
            o = aq / jnp.maximum(cq, 1.0) + ar / jnp.maximum(cr, 1.0) + fcb16
            obuf[pl.ds(j * 16, 16)] = o
        pltpu.sync_copy(obuf, out_hbm)


def _prep_body(x_ref, da_ref, db_ref, w1_ref, y0_ref, y1_ref, y2_ref, y3_ref,
               dv_ref):
    dinv = lax.rsqrt(da_ref[...] + db_ref[...] + 1.0)
    t = jnp.dot(x_ref[...], w1_ref[...], preferred_element_type=F32)
    y0_ref[...] = t[:, :32] * dinv
    y1_ref[...] = t[:, 32:64] * dinv
    y2_ref[...] = t[:, 64:96] * dinv
    y3_ref[...] = t[:, 96:] * dinv
    dv_ref[...] = dinv


def _stage1_body(s0_ref, s1_ref, s2_ref, s3_ref, y0_ref, y1_ref, y2_ref,
                 y3_ref, dv_ref, b1_ref, w2_ref, olo_ref, ohi_ref):
    dinv = dv_ref[...]
    h1 = jnp.concatenate(
        [dinv * (s0_ref[...] + y0_ref[...]),
         dinv * (s1_ref[...] + y1_ref[...]),
         dinv * (s2_ref[...] + y2_ref[...]),
         dinv * (s3_ref[...] + y3_ref[...])], axis=1)
    h1 = jnp.maximum(h1 + b1_ref[0], 0.0)
    t2 = jnp.dot(h1, w2_ref[...], preferred_element_type=F32)
    olo_ref[...] = t2[:, :32] * dinv
    ohi_ref[...] = t2[:, 32:] * dinv


def _stage2_body(slo_ref, shi_ref, ylo_ref, yhi_ref, dv_ref, b2_ref, w3_ref,
                 o_ref):
    dinv = dv_ref[...]
    zlo = dinv * (slo_ref[...] + ylo_ref[...])
    zhi = dinv * (shi_ref[...] + yhi_ref[...])
    h2 = jnp.maximum(jnp.concatenate([zlo, zhi], axis=1) + b2_ref[0], 0.0)
    o_ref[...] = jnp.dot(h2, w3_ref[...], preferred_element_type=F32) * dinv


def _stage3_body(sa_ref, sb_ref, y3_ref, dv_ref, b3_ref, fcw_ref, z_ref):
    h3 = jnp.maximum(
        dv_ref[...] * (sa_ref[...] + sb_ref[...] + y3_ref[...]) + b3_ref[0],
        0.0)
    z = jnp.dot(h3, fcw_ref[...].reshape(32, 1), preferred_element_type=F32)
    z_ref[...] = jnp.broadcast_to(z, (RB, 32))


def _nspec(w=32):
    return pl.BlockSpec((RB, w), lambda i: (i, 0))


def _fspec(shape):
    return pl.BlockSpec(shape, lambda i: tuple(0 for _ in shape))


_prep = pl.pallas_call(
    _prep_body,
    grid=(NBLK,),
    in_specs=[_nspec(64), _nspec(), _nspec(), _fspec((64, 128))],
    out_specs=[_nspec(), _nspec(), _nspec(), _nspec(), _nspec()],
    out_shape=[_NSD, _NSD, _NSD, _NSD, _NSD],
)

_stage1 = pl.pallas_call(
    _stage1_body,
    grid=(NBLK,),
    in_specs=[_nspec(), _nspec(), _nspec(), _nspec(),
              _nspec(), _nspec(), _nspec(), _nspec(), _nspec(),
              _fspec((1, 128)), _fspec((128, 64))],
    out_specs=[_nspec(), _nspec()],
    out_shape=[_NSD, _NSD],
)

_stage2 = pl.pallas_call(
    _stage2_body,
    grid=(NBLK,),
    in_specs=[_nspec(), _nspec(), _nspec(), _nspec(), _nspec(),
              _fspec((1, 64)), _fspec((64, 32))],
    out_specs=_nspec(),
    out_shape=_NSD,
)

_stage3 = pl.pallas_call(
    _stage3_body,
    grid=(NBLK,),
    in_specs=[_nspec(), _nspec(), _nspec(), _nspec(),
              _fspec((1, 32)), _fspec((1, 32))],
    out_specs=_nspec(),
    out_shape=_NSD,
)


def _pad_nodes(x):
    return jnp.pad(x, ((0, NP - N), (0, 0)))


def _pad_edges(e):
    return jnp.pad(e, (0, EPAD - E), constant_values=NP - 1)


def _encode_graph(x, src, dst, zblk, onesb, W1, b1, W2, b2, W3, b3, fcw):
    da, db = _deg_g(dst, onesb, zblk)
    yq0, yq1, yq2, yq3, dv = _prep(_pad_nodes(x), da, db, W1)
    sq0, sq1, sq2, sq3 = _agg4_g(yq0, yq1, yq2, yq3, src, dst, zblk)
    y2lo, y2hi = _stage1(sq0, sq1, sq2, sq3, yq0, yq1, yq2, yq3, dv,
                         b1.reshape(1, 128), W2)
    s2lo, s2hi = _agg2_g(y2lo, y2hi, src, dst, zblk)
    y3 = _stage2(s2lo, s2hi, y2lo, y2hi, dv, b2.reshape(1, 64), W3)
    s3a, s3b = _agg1_g(y3, src, dst, zblk)
    return _stage3(s3a, s3b, y3, dv, b3.reshape(1, 32), fcw.reshape(1, 32))


def kernel(x_q, edge_index_q, batch_q, x_r, edge_index_r, batch_r,
           W1, b1, W2, b2, W3, b3, fcW, fcb):
    ZBLK = jnp.zeros((TILE_N, 32), F32)
    ONESB = jnp.ones((EBLK, 32), F32)
    FCB = jnp.broadcast_to(fcb, (16,))
    BQ = jnp.pad(batch_q, (0, NP - N), constant_values=G)
    BR = jnp.pad(batch_r, (0, NP - N), constant_values=G)
    zbq = _encode_graph(x_q, _pad_edges(edge_index_q[0]),
                        _pad_edges(edge_index_q[1]), ZBLK, ONESB,
                        W1, b1, W2, b2, W3, b3, fcW[:32])
    zbr = _encode_graph(x_r, _pad_edges(edge_index_r[0]),
                        _pad_edges(edge_index_r[1]), ZBLK, ONESB,
                        W1, b1, W2, b2, W3, b3, fcW[32:])
    return _pool_kernel(zbq, zbr, BQ, BR, FCB)


# speedup vs baseline: 1.3207x; 1.3207x over previous
import functools

import jax
import jax.numpy as jnp
from jax import lax
from jax.experimental import pallas as pl
from jax.experimental.pallas import tpu as pltpu
from jax.experimental.pallas import tpu_sc as plsc

N = 50000
E = 800000
G = 64
GP = 80
NP = 50176
TILE_N = NP // 16
EB = 6272
EBLK = 128
EPAD = EB * EBLK
NBT_ALL = EB // 16
NBT_SPL = EB // 32
DPIPE = 4
RB = 1568
NBLK = NP // RB
F32 = jnp.float32

_MESH = plsc.VectorSubcoreMesh(core_axis_name="c", subcore_axis_name="s")
_SC_PARAMS = pltpu.CompilerParams(needs_layout_passes=False,
                                  use_tc_tiling_on_sc=False)
_NSD = jax.ShapeDtypeStruct((NP, 32), F32)


def _edge_pipeline(idx_start, idx_wait, work, drain_one, nblocks):
    ngrp = nblocks // DPIPE
    for b in range(DPIPE):
        idx_start(b, b)

    def grp(g, carry):
        for b in range(DPIPE):
            idx_wait(b)
            work(b)

        def drain(b, carry2):
            drain_one(b)

            @pl.when(g < ngrp - 1)
            def _():
                idx_start(g * DPIPE + DPIPE + b, b)
            return carry2
        lax.fori_loop(0, DPIPE, drain, 0)
        return carry
    lax.fori_loop(0, ngrp, grp, 0)


@functools.partial(
    pl.kernel,
    out_type=[_NSD, _NSD],
    mesh=_MESH,
    compiler_params=_SC_PARAMS,
    scratch_types=[
        pltpu.VMEM((DPIPE, EBLK), jnp.int32),
        pltpu.VMEM((EBLK, 32), F32),
        pltpu.VMEM_SHARED((NP, 32), F32),
        pltpu.SemaphoreType.DMA,
        pltpu.SemaphoreType.DMA,
    ],
)
def _deg_g(dst_hbm, onesb_hbm, zblk_hbm, oa, ob, dstb, ones, acc, isem, ssem):
    c = lax.axis_index("c")
    s = lax.axis_index("s")
    pltpu.sync_copy(onesb_hbm, ones)
    pltpu.sync_copy(zblk_hbm, acc.at[pl.ds(s * TILE_N, TILE_N)])
    plsc.subcore_barrier()

    def idx_start(j, b):
        pltpu.async_copy(
            dst_hbm.at[pl.ds(((c * 16 + s) * NBT_SPL + j) * EBLK, EBLK)],
            dstb.at[b], isem)

    def idx_wait(b):
        pltpu.make_async_copy(dst_hbm.at[pl.ds(0, EBLK)],
                              dstb.at[b], isem).wait()

    def work(b):
        pltpu.async_copy(ones, acc.at[dstb.at[b]], ssem, add=True)

    def drain_one(b):
        pltpu.make_async_copy(onesb_hbm, ones, ssem).wait()

    _edge_pipeline(idx_start, idx_wait, work, drain_one, NBT_SPL)
    plsc.subcore_barrier()
    for cc, out in ((0, oa), (1, ob)):
        @pl.when(c == cc)
        def _(out=out):
            pltpu.sync_copy(acc.at[pl.ds(s * TILE_N, TILE_N)],
                            out.at[pl.ds(s * TILE_N, TILE_N)])


@functools.partial(
    pl.kernel,
    out_type=[_NSD, _NSD],
    mesh=_MESH,
    compiler_params=_SC_PARAMS,
    scratch_types=[
        pltpu.VMEM((DPIPE, EBLK), jnp.int32),
        pltpu.VMEM((DPIPE, EBLK), jnp.int32),
        pltpu.VMEM((DPIPE, EBLK, 32), F32),
        pltpu.VMEM_SHARED((NP, 32), F32),
        pltpu.SemaphoreType.DMA,
        pltpu.SemaphoreType.DMA,
        pltpu.SemaphoreType.DMA,
    ],
)
def _agg2_g(ylo_hbm, yhi_hbm, src_hbm, dst_hbm, zblk_hbm, olo, ohi,
            srcb, dstb, rows, acc, isem, gsem, ssem):
    c = lax.axis_index("c")
    s = lax.axis_index("s")
    pltpu.sync_copy(zblk_hbm, acc.at[pl.ds(s * TILE_N, TILE_N)])
    plsc.subcore_barrier()

    def idx_start(j, b):
        base = (s * NBT_ALL + j) * EBLK
        pltpu.async_copy(src_hbm.at[pl.ds(base, EBLK)], srcb.at[b], isem)
        pltpu.async_copy(dst_hbm.at[pl.ds(base, EBLK)], dstb.at[b], isem)

    def idx_wait(b):
        pltpu.make_async_copy(src_hbm.at[pl.ds(0, EBLK)],
                              srcb.at[b], isem).wait()
        pltpu.make_async_copy(dst_hbm.at[pl.ds(0, EBLK)],
                              dstb.at[b], isem).wait()

    def drain_one(b):
        pltpu.make_async_copy(ylo_hbm.at[pl.ds(0, EBLK)],
                              rows.at[b], ssem).wait()

    for h, y, out in ((0, ylo_hbm, olo), (1, yhi_hbm, ohi)):
        @pl.when(c == h)
        def _(y=y, out=out):
            def work(b):
                pltpu.async_copy(y.at[srcb.at[b]], rows.at[b], gsem)

                def scat(bb, carry):
                    pltpu.make_async_copy(y.at[pl.ds(0, EBLK)],
                                          rows.at[bb], gsem).wait()
                    pltpu.async_copy(rows.at[bb], acc.at[dstb.at[bb]], ssem,
                                     add=True)
                    return carry
                if b == DPIPE - 1:
                    lax.fori_loop(0, DPIPE, scat, 0)

            _edge_pipeline(idx_start, idx_wait, work, drain_one, NBT_ALL)
            plsc.subcore_barrier()
            pltpu.sync_copy(acc.at[pl.ds(s * TILE_N, TILE_N)],
                            out.at[pl.ds(s * TILE_N, TILE_N)])


@functools.partial(
    pl.kernel,
    out_type=[_NSD, _NSD],
    mesh=_MESH,
    compiler_params=_SC_PARAMS,
    scratch_types=[
        pltpu.VMEM((DPIPE, EBLK), jnp.int32),
        pltpu.VMEM((DPIPE, EBLK), jnp.int32),
        pltpu.VMEM((DPIPE, EBLK, 32), F32),
        pltpu.VMEM_SHARED((NP, 32), F32),
        pltpu.SemaphoreType.DMA,
        pltpu.SemaphoreType.DMA,
        pltpu.SemaphoreType.DMA,
    ],
)
def _agg1_g(y_hbm, src_hbm, dst_hbm, zblk_hbm, oa, ob,
            srcb, dstb, rows, acc, isem, gsem, ssem):
    c = lax.axis_index("c")
    s = lax.axis_index("s")
    pltpu.sync_copy(zblk_hbm, acc.at[pl.ds(s * TILE_N, TILE_N)])
    plsc.subcore_barrier()

    def idx_start(j, b):
        base = ((c * 16 + s) * NBT_SPL + j) * EBLK
        pltpu.async_copy(src_hbm.at[pl.ds(base, EBLK)], srcb.at[b], isem)
        pltpu.async_copy(dst_hbm.at[pl.ds(base, EBLK)], dstb.at[b], isem)

    def idx_wait(b):
        pltpu.make_async_copy(src_hbm.at[pl.ds(0, EBLK)],
                              srcb.at[b], isem).wait()
        pltpu.make_async_copy(dst_hbm.at[pl.ds(0, EBLK)],
                              dstb.at[b], isem).wait()

    def work(b):
        pltpu.async_copy(y_hbm.at[srcb.at[b]], rows.at[b], gsem)

        def scat(bb, carry):
            pltpu.make_async_copy(y_hbm.at[pl.ds(0, EBLK)],
                                  rows.at[bb], gsem).wait()
            pltpu.async_copy(rows.at[bb], acc.at[dstb.at[bb]], ssem, add=True)
            return carry
        if b == DPIPE - 1:
            lax.fori_loop(0, DPIPE, scat, 0)

    def drain_one(b):
        pltpu.make_async_copy(y_hbm.at[pl.ds(0, EBLK)],
                              rows.at[b], ssem).wait()

    _edge_pipeline(idx_start, idx_wait, work, drain_one, NBT_SPL)
    plsc.subcore_barrier()
    for cc, out in ((0, oa), (1, ob)):
        @pl.when(c == cc)
        def _(out=out):
            pltpu.sync_copy(acc.at[pl.ds(s * TILE_N, TILE_N)],
                            out.at[pl.ds(s * TILE_N, TILE_N)])


_PR = NP // 8
_CHZ = 448
_NCHZ = _PR // _CHZ


@functools.partial(
    pl.kernel,
    out_type=jax.ShapeDtypeStruct((G,), F32),
    mesh=_MESH,
    compiler_params=_SC_PARAMS,
    scratch_types=[
        pltpu.VMEM((_CHZ, 32), F32),
        pltpu.VMEM((_PR,), jnp.int32),
        pltpu.VMEM((GP,), F32),
        pltpu.VMEM((GP,), F32),
        pltpu.VMEM((16 * 2 * GP,), F32),
        pltpu.VMEM((G,), F32),
        pltpu.VMEM((16,), F32),
        pltpu.VMEM_SHARED((16 * 2 * GP,), F32),
    ],
)
def _pool_kernel(zq_hbm, zr_hbm, bq_hbm, br_hbm, fcb_hbm, out_hbm,
                 zbuf, bbuf, accl, cntl, slab, obuf, fbuf, stage):
    c = lax.axis_index("c")
    s = lax.axis_index("s")
    gg = lax.rem(s, 2)
    ci = lax.div(s, 2)
    zero16 = jnp.zeros((16,), F32)
    ones16 = jnp.ones((16,), F32)
    iota16 = lax.iota(jnp.int32, 16)
    zero16i = jnp.zeros((16,), jnp.int32)
    for gch, bh in ((0, bq_hbm), (1, br_hbm)):
        @pl.when(gg == gch)
        def _(bh=bh):
            pltpu.sync_copy(bh.at[pl.ds(ci * _PR, _PR)], bbuf)
    pltpu.sync_copy(fcb_hbm, fbuf)
    for t in range(GP // 16):
        accl[pl.ds(t * 16, 16)] = zero16
        cntl[pl.ds(t * 16, 16)] = zero16

    def chunk(ch, carry):
        for gch, zh in ((0, zq_hbm), (1, zr_hbm)):
            @pl.when(gg == gch)
            def _(zh=zh):
                pltpu.sync_copy(zh.at[pl.ds(ci * _PR + ch * _CHZ, _CHZ)], zbuf)

        def it(k, carry2):
            b16 = bbuf[pl.ds(ch * _CHZ + k * 16, 16)]
            rows16 = iota16 + k * 16
            z16 = plsc.load_gather(zbuf, [rows16, zero16i])
            plsc.addupdate_scatter(accl, [b16], z16)
            plsc.addupdate_scatter(cntl, [b16], ones16)
            return carry2
        lax.fori_loop(0, _CHZ // 16, it, 0)
        return carry
    lax.fori_loop(0, _NCHZ, chunk, 0)

    pltpu.sync_copy(accl, stage.at[pl.ds(s * 2 * GP, GP)])
    pltpu.sync_copy(cntl, stage.at[pl.ds(s * 2 * GP + GP, GP)])
    plsc.subcore_barrier()

    @pl.when(jnp.logical_and(c == 0, s == 0))
    def _():
        pltpu.sync_copy(stage, slab)
        fcb16 = fbuf[...]
        for j in range(G // 16):
            def sl(r):
                return pl.ds(r * 2 * GP + j * 16, 16)

            def slc(r):
                return pl.ds(r * 2 * GP + GP + j * 16, 16)
            aq = slab[sl(0)]
            cq = slab[slc(0)]
            ar = slab[sl(1)]
            cr = slab[slc(1)]
            for r in range(2, 16, 2):
                aq = aq + slab[sl(r)]
                cq = cq + slab[slc(r)]
                ar = ar + slab[sl(r + 1)]
                cr = cr + slab[slc(r + 1)]
            o = aq / jnp.maximum(cq, 1.0) + ar / jnp.maximum(cr, 1.0) + fcb16
            obuf[pl.ds(j * 16, 16)] = o
        pltpu.sync_copy(obuf, out_hbm)


def _prep_body(x_ref, da_ref, db_ref, ylo_ref, yhi_ref, dv_ref):
    dinv = lax.rsqrt(da_ref[...] + db_ref[...] + 1.0)
    ylo_ref[...] = x_ref[:, :32] * dinv
    yhi_ref[...] = x_ref[:, 32:] * dinv
    dv_ref[...] = dinv


def _stage1_body(slo_ref, shi_ref, ylo_ref, yhi_ref, dv_ref, w1_ref, b1_ref,
                 w2_ref, olo_ref, ohi_ref):
    dinv = dv_ref[...]
    zlo = dinv * (slo_ref[...] + ylo_ref[...])
    zhi = dinv * (shi_ref[...] + yhi_ref[...])
    z = jnp.concatenate([zlo, zhi], axis=1)
    h1 = jnp.maximum(jnp.dot(z, w1_ref[...], preferred_element_type=F32)
                     + b1_ref[0], 0.0)
    t2 = jnp.dot(h1, w2_ref[...], preferred_element_type=F32)
    olo_ref[...] = t2[:, :32] * dinv
    ohi_ref[...] = t2[:, 32:] * dinv


def _stage2_body(slo_ref, shi_ref, ylo_ref, yhi_ref, dv_ref, b2_ref, w3_ref,
                 o_ref):
    dinv = dv_ref[...]
    zlo = dinv * (slo_ref[...] + ylo_ref[...])
    zhi = dinv * (shi_ref[...] + yhi_ref[...])
    h2 = jnp.maximum(jnp.concatenate([zlo, zhi], axis=1) + b2_ref[0], 0.0)
    o_ref[...] = jnp.dot(h2, w3_ref[...], preferred_element_type=F32) * dinv


def _stage3_body(sa_ref, sb_ref, y3_ref, dv_ref, b3_ref, fcw_ref, z_ref):
    h3 = jnp.maximum(
        dv_ref[...] * (sa_ref[...] + sb_ref[...] + y3_ref[...]) + b3_ref[0],
        0.0)
    z = jnp.dot(h3, fcw_ref[...].reshape(32, 1), preferred_element_type=F32)
    z_ref[...] = jnp.broadcast_to(z, (RB, 32))


def _nspec(w=32):
    return pl.BlockSpec((RB, w), lambda i: (i, 0))


def _fspec(shape):
    return pl.BlockSpec(shape, lambda i: tuple(0 for _ in shape))


_prep = pl.pallas_call(
    _prep_body,
    grid=(NBLK,),
    in_specs=[_nspec(64), _nspec(), _nspec()],
    out_specs=[_nspec(), _nspec(), _nspec()],
    out_shape=[_NSD, _NSD, _NSD],
)

_stage1 = pl.pallas_call(
    _stage1_body,
    grid=(NBLK,),
    in_specs=[_nspec(), _nspec(), _nspec(), _nspec(), _nspec(),
              _fspec((64, 128)), _fspec((1, 128)), _fspec((128, 64))],
    out_specs=[_nspec(), _nspec()],
    out_shape=[_NSD, _NSD],
)

_stage2 = pl.pallas_call(
    _stage2_body,
    grid=(NBLK,),
    in_specs=[_nspec(), _nspec(), _nspec(), _nspec(), _nspec(),
              _fspec((1, 64)), _fspec((64, 32))],
    out_specs=_nspec(),
    out_shape=_NSD,
)

_stage3 = pl.pallas_call(
    _stage3_body,
    grid=(NBLK,),
    in_specs=[_nspec(), _nspec(), _nspec(), _nspec(),
              _fspec((1, 32)), _fspec((1, 32))],
    out_specs=_nspec(),
    out_shape=_NSD,
)


def _pad_nodes(x):
    return jnp.pad(x, ((0, NP - N), (0, 0)))


def _pad_edges(e):
    return jnp.pad(e, (0, EPAD - E), constant_values=NP - 1)


def _encode_graph(x, src, dst, zblk, onesb, W1, b1, W2, b2, W3, b3, fcw):
    da, db = _deg_g(dst, onesb, zblk)
    ylo, yhi, dv = _prep(_pad_nodes(x), da, db)
    s1lo, s1hi = _agg2_g(ylo, yhi, src, dst, zblk)
    y2lo, y2hi = _stage1(s1lo, s1hi, ylo, yhi, dv, W1, b1.reshape(1, 128), W2)
    s2lo, s2hi = _agg2_g(y2lo, y2hi, src, dst, zblk)
    y3 = _stage2(s2lo, s2hi, y2lo, y2hi, dv, b2.reshape(1, 64), W3)
    s3a, s3b = _agg1_g(y3, src, dst, zblk)
    return _stage3(s3a, s3b, y3, dv, b3.reshape(1, 32), fcw.reshape(1, 32))


def kernel(x_q, edge_index_q, batch_q, x_r, edge_index_r, batch_r,
           W1, b1, W2, b2, W3, b3, fcW, fcb):
    ZBLK = jnp.zeros((TILE_N, 32), F32)
    ONESB = jnp.ones((EBLK, 32), F32)
    FCB = jnp.broadcast_to(fcb, (16,))
    BQ = jnp.pad(batch_q, (0, NP - N), constant_values=G)
    BR = jnp.pad(batch_r, (0, NP - N), constant_values=G)
    zbq = _encode_graph(x_q, _pad_edges(edge_index_q[0]),
                        _pad_edges(edge_index_q[1]), ZBLK, ONESB,
                        W1, b1, W2, b2, W3, b3, fcW[:32])
    zbr = _encode_graph(x_r, _pad_edges(edge_index_r[0]),
                        _pad_edges(edge_index_r[1]), ZBLK, ONESB,
                        W1, b1, W2, b2, W3, b3, fcW[32:])
    return _pool_kernel(zbq, zbr, BQ, BR, FCB)
